# baseline (speedup 1.0000x reference)
import functools

import jax
import jax.numpy as jnp
from jax import lax
from jax.experimental import pallas as pl
from jax.experimental.pallas import tpu as pltpu
from jax.experimental.pallas import tpu_sc as plsc

N_NODES = 10000
D = 256
N_EDGES = 160000

NC = 2
NS = 16
NW = NC * NS
C = 64
NCH = 80
PER_W = C * NCH
E_PAD = NW * PER_W

L = 16


def _make_sc_kernel():
  mesh = plsc.VectorSubcoreMesh(
      core_axis_name="c", subcore_axis_name="s", num_cores=NC, num_subcores=NS
  )

  @functools.partial(
      pl.kernel,
      out_type=jax.ShapeDtypeStruct((NW, NCH, C), jnp.float32),
      mesh=mesh,
      scratch_types=[
          pltpu.VMEM((NCH, C), jnp.int32),
          pltpu.VMEM((NCH, C), jnp.int32),
          pltpu.VMEM((NCH, C), jnp.float32),
          pltpu.VMEM((C, D), jnp.float32),
          pltpu.VMEM((C, D), jnp.float32),
          pltpu.VMEM((C, D), jnp.float32),
          pltpu.VMEM((C, D), jnp.float32),
          pltpu.SemaphoreType.DMA,
          pltpu.SemaphoreType.DMA,
      ],
      compiler_params=pltpu.CompilerParams(
          use_tc_tiling_on_sc=False, needs_layout_passes=False
      ),
  )
  def k(z_hbm, src_hbm, dst_hbm, out_hbm,
        src_v, dst_v, out_v, zs0, zs1, zd0, zd1, sem0, sem1):
    wid = lax.axis_index("s") * NC + lax.axis_index("c")
    zs_bufs = (zs0, zs1)
    zd_bufs = (zd0, zd1)
    sems = (sem0, sem1)

    pltpu.sync_copy(src_hbm.at[wid], src_v)
    pltpu.sync_copy(dst_hbm.at[wid], dst_v)

    def start(i, b):
      pltpu.async_copy(z_hbm.at[src_v.at[i]], zs_bufs[b], sems[b])
      pltpu.async_copy(z_hbm.at[dst_v.at[i]], zd_bufs[b], sems[b])

    def wait(i, b):
      pltpu.make_async_copy(z_hbm.at[src_v.at[i]], zs_bufs[b], sems[b]).wait()
      pltpu.make_async_copy(z_hbm.at[dst_v.at[i]], zd_bufs[b], sems[b]).wait()

    def compute(i, b):
      zs = zs_bufs[b]
      zd = zd_bufs[b]
      for g in range(C // L):
        rows = lax.iota(jnp.int32, L) + (g * L)

        def body(_, carry):
          acc, col = carry
          a = plsc.load_gather(zs, [rows, col])
          bb = plsc.load_gather(zd, [rows, col])
          return acc + a * bb, col + 1

        acc, _ = lax.fori_loop(
            0, D, body,
            (jnp.zeros((L,), jnp.float32), jnp.zeros((L,), jnp.int32)),
            unroll=8,
        )
        out_v[i, pl.ds(g * L, L)] = acc

    start(0, 0)

    def outer(i2, _):
      for b in range(2):
        i = i2 * 2 + b

        @pl.when(i + 1 < NCH)
        def _():
          start(i + 1, 1 - b)

        wait(i, b)
        compute(i, b)
      return ()

    lax.fori_loop(0, NCH // 2, outer, ())

    pltpu.sync_copy(out_v, out_hbm.at[wid])

  return k


_sc_kernel = _make_sc_kernel()


@jax.jit
def kernel(z, edge):
  e = edge.astype(jnp.int32)
  pad = jnp.zeros((E_PAD - N_EDGES, 2), jnp.int32)
  ep = jnp.concatenate([e, pad], axis=0)
  src = ep[:, 0].reshape(NW, NCH, C)
  dst = ep[:, 1].reshape(NW, NCH, C)
  out3 = _sc_kernel(z, src, dst)
  return out3.reshape(E_PAD)[:N_EDGES]


# speedup vs baseline: 2.3245x; 2.3245x over previous
import functools

import jax
import jax.numpy as jnp
from jax import lax
from jax.experimental import pallas as pl
from jax.experimental.pallas import tpu as pltpu
from jax.experimental.pallas import tpu_sc as plsc

N_NODES = 10000
D = 256
N_EDGES = 160000

NC = 2
NS = 16
NW = NC * NS
C = 64
NCH = 80
PER_W = C * NCH
E_PAD = NW * PER_W

L = 16


def _make_sc_kernel():
  mesh = plsc.VectorSubcoreMesh(
      core_axis_name="c", subcore_axis_name="s", num_cores=NC, num_subcores=NS
  )

  @functools.partial(
      pl.kernel,
      out_type=jax.ShapeDtypeStruct((NW, NCH, C), jnp.float32),
      mesh=mesh,
      scratch_types=[
          pltpu.VMEM((NCH, C), jnp.int32),
          pltpu.VMEM((NCH, C), jnp.int32),
          pltpu.VMEM((NCH, C), jnp.float32),
          pltpu.VMEM((C, D), jnp.float32),
          pltpu.VMEM((C, D), jnp.float32),
          pltpu.VMEM((C, D), jnp.float32),
          pltpu.VMEM((C, D), jnp.float32),
          pltpu.SemaphoreType.DMA,
          pltpu.SemaphoreType.DMA,
      ],
      compiler_params=pltpu.CompilerParams(
          use_tc_tiling_on_sc=False, needs_layout_passes=False
      ),
  )
  def k(z_hbm, src_hbm, dst_hbm, out_hbm,
        src_v, dst_v, out_v, zs0, zs1, zd0, zd1, sem0, sem1):
    wid = lax.axis_index("s") * NC + lax.axis_index("c")
    zs_bufs = (zs0, zs1)
    zd_bufs = (zd0, zd1)
    sems = (sem0, sem1)

    pltpu.sync_copy(src_hbm.at[wid], src_v)
    pltpu.sync_copy(dst_hbm.at[wid], dst_v)

    def start(i, b):
      pltpu.async_copy(z_hbm.at[src_v.at[i]], zs_bufs[b], sems[b])
      pltpu.async_copy(z_hbm.at[dst_v.at[i]], zd_bufs[b], sems[b])

    def wait(i, b):
      pltpu.make_async_copy(z_hbm.at[src_v.at[i]], zs_bufs[b], sems[b]).wait()
      pltpu.make_async_copy(z_hbm.at[dst_v.at[i]], zd_bufs[b], sems[b]).wait()

    NU = 8

    def compute(i, b):
      zs = zs_bufs[b]
      zd = zd_bufs[b]
      for g in range(C // L):
        rows = lax.iota(jnp.int32, L) + (g * L)
        col0 = lax.iota(jnp.int32, L)

        def body(_, carry):
          accs, colb = carry
          new = []
          for s in range(NU):
            cs = (colb + s) & (D - 1)
            a = plsc.load_gather(zs, [rows, cs])
            bb = plsc.load_gather(zd, [rows, cs])
            new.append(accs[s] + a * bb)
          return tuple(new), colb + NU

        accs, _ = lax.fori_loop(
            0, D // NU, body,
            (tuple(jnp.zeros((L,), jnp.float32) for _ in range(NU)), col0),
            unroll=4,
        )
        acc = ((accs[0] + accs[1]) + (accs[2] + accs[3])) + (
            (accs[4] + accs[5]) + (accs[6] + accs[7]))
        out_v[i, pl.ds(g * L, L)] = acc

    start(0, 0)

    def outer(i2, _):
      for b in range(2):
        i = i2 * 2 + b

        @pl.when(i + 1 < NCH)
        def _():
          start(i + 1, 1 - b)

        wait(i, b)
        compute(i, b)
      return ()

    lax.fori_loop(0, NCH // 2, outer, ())

    pltpu.sync_copy(out_v, out_hbm.at[wid])

  return k


_sc_kernel = _make_sc_kernel()


@jax.jit
def kernel(z, edge):
  e = edge.astype(jnp.int32)
  pad = jnp.zeros((E_PAD - N_EDGES, 2), jnp.int32)
  ep = jnp.concatenate([e, pad], axis=0)
  src = ep[:, 0].reshape(NW, NCH, C)
  dst = ep[:, 1].reshape(NW, NCH, C)
  out3 = _sc_kernel(z, src, dst)
  return out3.reshape(E_PAD)[:N_EDGES]
